# baseline (device time: 78302 ns/iter reference)
import jax
import jax.numpy as jnp
from jax import lax
from jax.experimental import pallas as pl
from jax.experimental.pallas import tpu as pltpu

T = 512
D = 1024
V_LOC = 8192
KS = 16
SW = V_LOC // KS
KC = 8
CW = V_LOC // KC


def kernel(x, W):
    def body(x_ref, w_hbm, out_ref, x_bf, w_buf, my_stats, peer_stats,
             w_sems, sy_sems, f_sems, ry_sems, fr_sems, st_sems):
        my_x = lax.axis_index("x")
        my_y = lax.axis_index("y")
        xn = (1 - my_x, my_y)
        yn = (my_x, 1 - my_y)

        my_col0 = my_y * V_LOC
        other_col0 = (1 - my_y) * V_LOC

        def rcopy(src, dst, ssem, rsem, dev):
            return pltpu.make_async_remote_copy(
                src_ref=src, dst_ref=dst, send_sem=ssem, recv_sem=rsem,
                device_id=dev, device_id_type=pl.DeviceIdType.MESH,
            )

        barrier_sem = pltpu.get_barrier_semaphore()
        for nbr in (xn, yn):
            pl.semaphore_signal(
                barrier_sem, inc=1, device_id=nbr,
                device_id_type=pl.DeviceIdType.MESH,
            )

        x_bf[...] = x_ref[...].astype(jnp.bfloat16)

        def w_dma(j):
            return pltpu.make_async_copy(
                w_hbm.at[:, pl.ds(j * SW, SW)],
                w_buf.at[j % 2],
                w_sems.at[j % 2],
            )

        w_dma(0).start()
        sys_ = [None] * KC
        for j in range(KS):
            if j + 1 < KS:
                w_dma(j + 1).start()
            w_dma(j).wait()
            wj = w_buf[j % 2].astype(jnp.bfloat16)
            logits = jnp.dot(
                x_bf[...], wj, preferred_element_type=jnp.float32
            )
            m = logits.max(axis=1, keepdims=True)
            e = jnp.exp(logits - m)
            my_stats[0, :, j : j + 1] = m
            my_stats[1, :, j : j + 1] = e.sum(axis=1, keepdims=True)
            out_ref[:, pl.ds(my_col0 + j * SW, SW)] = e.astype(jnp.bfloat16)
            if j == 0:
                pl.semaphore_wait(barrier_sem, 2)
            if j % 2 == 1:
                k = j // 2
                blk = out_ref.at[:, pl.ds(my_col0 + k * CW, CW)]
                send = rcopy(blk, blk, sy_sems.at[k], ry_sems.at[k], yn)
                @pl.when(k % 2 == my_x)
                def _():
                    send.start()
                sys_[k] = send

        st = rcopy(my_stats, peer_stats, st_sems.at[0], st_sems.at[1], yn)
        st.start()

        fwds = [None] * KC
        recvs = [None] * KC
        for k in range(KC):
            blk = out_ref.at[:, pl.ds(other_col0 + k * CW, CW)]
            direct = rcopy(blk, blk, sy_sems.at[k], ry_sems.at[k], yn)
            fwd = rcopy(blk, blk, f_sems.at[k], fr_sems.at[k], xn)
            recvs[k] = rcopy(blk, blk, f_sems.at[k], fr_sems.at[k], xn)

            @pl.when(k % 2 == my_x)
            def _():
                direct.wait_recv()
                fwd.start()

            fwds[k] = fwd

        rcopy(my_stats, peer_stats, st_sems.at[0], st_sems.at[1], yn).wait_recv()

        mm, ms = my_stats[0], my_stats[1]
        pm, ps = peer_stats[0], peer_stats[1]
        m_fin = jnp.maximum(
            mm.max(axis=1, keepdims=True), pm.max(axis=1, keepdims=True)
        )
        em = jnp.exp(mm - m_fin)
        ep = jnp.exp(pm - m_fin)
        s_fin = (ms * em).sum(axis=1, keepdims=True) + (ep * ps).sum(
            axis=1, keepdims=True
        )
        inv = 1.0 / s_fin
        fac_mine = (em * inv).astype(jnp.bfloat16)
        fac_peer = (ep * inv).astype(jnp.bfloat16)

        def rescale(col0, fac, j):
            sl = pl.ds(col0 + j * SW, SW)
            out_ref[:, sl] = out_ref[:, sl] * fac[:, j : j + 1]

        for k in range(KC):
            send = sys_[k]

            @pl.when(k % 2 == my_x)
            def _():
                send.wait_send()

            for j in (2 * k, 2 * k + 1):
                rescale(my_col0, fac_mine, j)

        for k in range(KC):
            @pl.when(k % 2 == my_x)
            def _():
                fwds[k].wait_send()

            @pl.when(k % 2 != my_x)
            def _():
                recvs[k].wait_recv()

            for j in (2 * k, 2 * k + 1):
                rescale(other_col0, fac_peer, j)

        st.wait_send()

    stat_shape = pltpu.VMEM((2, T, KS), jnp.float32)
    return pl.pallas_call(
        body,
        out_shape=jax.ShapeDtypeStruct((T, 2 * V_LOC), jnp.bfloat16),
        in_specs=[
            pl.BlockSpec(memory_space=pltpu.VMEM),
            pl.BlockSpec(memory_space=pl.ANY),
        ],
        out_specs=pl.BlockSpec(memory_space=pltpu.VMEM),
        scratch_shapes=[
            pltpu.VMEM((T, D), jnp.bfloat16),
            pltpu.VMEM((2, D, SW), jnp.float32),
            stat_shape,
            stat_shape,
            pltpu.SemaphoreType.DMA((2,)),
            pltpu.SemaphoreType.DMA((KC,)),
            pltpu.SemaphoreType.DMA((KC,)),
            pltpu.SemaphoreType.DMA((KC,)),
            pltpu.SemaphoreType.DMA((KC,)),
            pltpu.SemaphoreType.DMA((2,)),
        ],
        compiler_params=pltpu.CompilerParams(collective_id=0),
    )(x, W)


# device time: 78296 ns/iter; 1.0001x vs baseline; 1.0001x over previous
import jax
import jax.numpy as jnp
from jax import lax
from jax.experimental import pallas as pl
from jax.experimental.pallas import tpu as pltpu

T = 512
D = 1024
V_LOC = 8192
KS = 16
SW = V_LOC // KS
KC = 16
CW = V_LOC // KC


def kernel(x, W):
    def body(x_ref, w_hbm, out_ref, x_bf, w_buf, my_stats, peer_stats,
             w_sems, sy_sems, f_sems, ry_sems, fr_sems, st_sems):
        my_x = lax.axis_index("x")
        my_y = lax.axis_index("y")
        xn = (1 - my_x, my_y)
        yn = (my_x, 1 - my_y)

        my_col0 = my_y * V_LOC
        other_col0 = (1 - my_y) * V_LOC

        def rcopy(src, dst, ssem, rsem, dev):
            return pltpu.make_async_remote_copy(
                src_ref=src, dst_ref=dst, send_sem=ssem, recv_sem=rsem,
                device_id=dev, device_id_type=pl.DeviceIdType.MESH,
            )

        barrier_sem = pltpu.get_barrier_semaphore()
        for nbr in (xn, yn):
            pl.semaphore_signal(
                barrier_sem, inc=1, device_id=nbr,
                device_id_type=pl.DeviceIdType.MESH,
            )

        x_bf[...] = x_ref[...].astype(jnp.bfloat16)

        def w_dma(j):
            return pltpu.make_async_copy(
                w_hbm.at[:, pl.ds(j * SW, SW)],
                w_buf.at[j % 2],
                w_sems.at[j % 2],
            )

        w_dma(0).start()
        sys_ = [None] * KC
        for j in range(KS):
            if j + 1 < KS:
                w_dma(j + 1).start()
            w_dma(j).wait()
            wj = w_buf[j % 2].astype(jnp.bfloat16)
            logits = jnp.dot(
                x_bf[...], wj, preferred_element_type=jnp.float32
            )
            m = logits.max(axis=1, keepdims=True)
            e = jnp.exp(logits - m)
            my_stats[0, :, j : j + 1] = m
            my_stats[1, :, j : j + 1] = e.sum(axis=1, keepdims=True)
            out_ref[:, pl.ds(my_col0 + j * SW, SW)] = e.astype(jnp.bfloat16)
            if j == 0:
                pl.semaphore_wait(barrier_sem, 2)
            R = KS // KC
            if j % R == R - 1:
                k = j // R
                blk = out_ref.at[:, pl.ds(my_col0 + k * CW, CW)]
                send = rcopy(blk, blk, sy_sems.at[k], ry_sems.at[k], yn)
                @pl.when(k % 2 == my_x)
                def _():
                    send.start()
                sys_[k] = send

        st = rcopy(my_stats, peer_stats, st_sems.at[0], st_sems.at[1], yn)
        st.start()

        fwds = [None] * KC
        recvs = [None] * KC
        for k in range(KC):
            blk = out_ref.at[:, pl.ds(other_col0 + k * CW, CW)]
            direct = rcopy(blk, blk, sy_sems.at[k], ry_sems.at[k], yn)
            fwd = rcopy(blk, blk, f_sems.at[k], fr_sems.at[k], xn)
            recvs[k] = rcopy(blk, blk, f_sems.at[k], fr_sems.at[k], xn)

            @pl.when(k % 2 == my_x)
            def _():
                direct.wait_recv()
                fwd.start()

            fwds[k] = fwd

        rcopy(my_stats, peer_stats, st_sems.at[0], st_sems.at[1], yn).wait_recv()

        mm, ms = my_stats[0], my_stats[1]
        pm, ps = peer_stats[0], peer_stats[1]
        m_fin = jnp.maximum(
            mm.max(axis=1, keepdims=True), pm.max(axis=1, keepdims=True)
        )
        em = jnp.exp(mm - m_fin)
        ep = jnp.exp(pm - m_fin)
        s_fin = (ms * em).sum(axis=1, keepdims=True) + (ep * ps).sum(
            axis=1, keepdims=True
        )
        inv = 1.0 / s_fin
        fac_mine = (em * inv).astype(jnp.bfloat16)
        fac_peer = (ep * inv).astype(jnp.bfloat16)

        def rescale(col0, fac, j):
            sl = pl.ds(col0 + j * SW, SW)
            out_ref[:, sl] = out_ref[:, sl] * fac[:, j : j + 1]

        for k in range(KC):
            send = sys_[k]

            @pl.when(k % 2 == my_x)
            def _():
                send.wait_send()

            for j in range((KS // KC) * k, (KS // KC) * (k + 1)):
                rescale(my_col0, fac_mine, j)

        for k in range(KC):
            @pl.when(k % 2 == my_x)
            def _():
                fwds[k].wait_send()

            @pl.when(k % 2 != my_x)
            def _():
                recvs[k].wait_recv()

            for j in range((KS // KC) * k, (KS // KC) * (k + 1)):
                rescale(other_col0, fac_peer, j)

        st.wait_send()

    stat_shape = pltpu.VMEM((2, T, KS), jnp.float32)
    return pl.pallas_call(
        body,
        out_shape=jax.ShapeDtypeStruct((T, 2 * V_LOC), jnp.bfloat16),
        in_specs=[
            pl.BlockSpec(memory_space=pltpu.VMEM),
            pl.BlockSpec(memory_space=pl.ANY),
        ],
        out_specs=pl.BlockSpec(memory_space=pltpu.VMEM),
        scratch_shapes=[
            pltpu.VMEM((T, D), jnp.bfloat16),
            pltpu.VMEM((2, D, SW), jnp.float32),
            stat_shape,
            stat_shape,
            pltpu.SemaphoreType.DMA((2,)),
            pltpu.SemaphoreType.DMA((KC,)),
            pltpu.SemaphoreType.DMA((KC,)),
            pltpu.SemaphoreType.DMA((KC,)),
            pltpu.SemaphoreType.DMA((KC,)),
            pltpu.SemaphoreType.DMA((2,)),
        ],
        compiler_params=pltpu.CompilerParams(collective_id=0),
    )(x, W)


# device time: 78255 ns/iter; 1.0006x vs baseline; 1.0005x over previous
import jax
import jax.numpy as jnp
from jax import lax
from jax.experimental import pallas as pl
from jax.experimental.pallas import tpu as pltpu

T = 512
D = 1024
V_LOC = 8192
KS = 16
SW = V_LOC // KS
KC = 8
CW = V_LOC // KC


def kernel(x, W):
    def body(x_ref, w_hbm, out_ref, x_bf, w_buf, my_stats, peer_stats,
             w_sems, sy_sems, f_sems, ry_sems, fr_sems, st_sems):
        my_x = lax.axis_index("x")
        my_y = lax.axis_index("y")
        xn = (1 - my_x, my_y)
        yn = (my_x, 1 - my_y)

        my_col0 = my_y * V_LOC
        other_col0 = (1 - my_y) * V_LOC

        def rcopy(src, dst, ssem, rsem, dev):
            return pltpu.make_async_remote_copy(
                src_ref=src, dst_ref=dst, send_sem=ssem, recv_sem=rsem,
                device_id=dev, device_id_type=pl.DeviceIdType.MESH,
            )

        barrier_sem = pltpu.get_barrier_semaphore()
        for nbr in (xn, yn):
            pl.semaphore_signal(
                barrier_sem, inc=1, device_id=nbr,
                device_id_type=pl.DeviceIdType.MESH,
            )

        x_bf[...] = x_ref[...].astype(jnp.bfloat16)

        def w_dma(j):
            return pltpu.make_async_copy(
                w_hbm.at[:, pl.ds(j * SW, SW)],
                w_buf.at[j % 2],
                w_sems.at[j % 2],
            )

        w_dma(0).start()
        sys_ = [None] * KC
        for j in range(KS):
            if j + 1 < KS:
                w_dma(j + 1).start()
            w_dma(j).wait()
            wj = w_buf[j % 2].astype(jnp.bfloat16)
            logits = jnp.dot(
                x_bf[...], wj, preferred_element_type=jnp.float32
            )
            m = logits.max(axis=1, keepdims=True)
            e = jnp.exp(logits - m)
            my_stats[0, :, j : j + 1] = m
            my_stats[1, :, j : j + 1] = e.sum(axis=1, keepdims=True)
            out_ref[:, pl.ds(my_col0 + j * SW, SW)] = e.astype(jnp.bfloat16)
            if j == 0:
                pl.semaphore_wait(barrier_sem, 2)
            if j % 2 == 1:
                k = j // 2
                blk = out_ref.at[:, pl.ds(my_col0 + k * CW, CW)]
                send = rcopy(blk, blk, sy_sems.at[k], ry_sems.at[k], yn)
                @pl.when(k % 2 == my_x)
                def _():
                    send.start()
                sys_[k] = send

        st = rcopy(my_stats, peer_stats, st_sems.at[0], st_sems.at[1], yn)
        st.start()

        fwds = [None] * KC
        recvs = [None] * KC
        for k in range(KC):
            blk = out_ref.at[:, pl.ds(other_col0 + k * CW, CW)]
            direct = rcopy(blk, blk, sy_sems.at[k], ry_sems.at[k], yn)
            fwd = rcopy(blk, blk, f_sems.at[k], fr_sems.at[k], xn)
            recvs[k] = rcopy(blk, blk, f_sems.at[k], fr_sems.at[k], xn)

            @pl.when(k % 2 == my_x)
            def _():
                direct.wait_recv()
                fwd.start()

            fwds[k] = fwd

        rcopy(my_stats, peer_stats, st_sems.at[0], st_sems.at[1], yn).wait_recv()

        mm, ms = my_stats[0], my_stats[1]
        pm, ps = peer_stats[0], peer_stats[1]
        m_fin = jnp.maximum(
            mm.max(axis=1, keepdims=True), pm.max(axis=1, keepdims=True)
        )
        em = jnp.exp(mm - m_fin)
        ep = jnp.exp(pm - m_fin)
        s_fin = (ms * em).sum(axis=1, keepdims=True) + (ep * ps).sum(
            axis=1, keepdims=True
        )
        inv = 1.0 / s_fin
        fac_mine = em * inv
        fac_peer = ep * inv

        def rescale(col0, fac, j):
            sl = pl.ds(col0 + j * SW, SW)
            out_ref[:, sl] = (
                out_ref[:, sl].astype(jnp.float32) * fac[:, j : j + 1]
            ).astype(jnp.bfloat16)

        for k in range(KC):
            send = sys_[k]

            @pl.when(k % 2 == my_x)
            def _():
                send.wait_send()

            for j in (2 * k, 2 * k + 1):
                rescale(my_col0, fac_mine, j)

        for k in range(KC):
            @pl.when(k % 2 == my_x)
            def _():
                fwds[k].wait_send()

            @pl.when(k % 2 != my_x)
            def _():
                recvs[k].wait_recv()

            for j in (2 * k, 2 * k + 1):
                rescale(other_col0, fac_peer, j)

        st.wait_send()

    stat_shape = pltpu.VMEM((2, T, KS), jnp.float32)
    return pl.pallas_call(
        body,
        out_shape=jax.ShapeDtypeStruct((T, 2 * V_LOC), jnp.bfloat16),
        in_specs=[
            pl.BlockSpec(memory_space=pltpu.VMEM),
            pl.BlockSpec(memory_space=pl.ANY),
        ],
        out_specs=pl.BlockSpec(memory_space=pltpu.VMEM),
        scratch_shapes=[
            pltpu.VMEM((T, D), jnp.bfloat16),
            pltpu.VMEM((2, D, SW), jnp.float32),
            stat_shape,
            stat_shape,
            pltpu.SemaphoreType.DMA((2,)),
            pltpu.SemaphoreType.DMA((KC,)),
            pltpu.SemaphoreType.DMA((KC,)),
            pltpu.SemaphoreType.DMA((KC,)),
            pltpu.SemaphoreType.DMA((KC,)),
            pltpu.SemaphoreType.DMA((2,)),
        ],
        compiler_params=pltpu.CompilerParams(collective_id=0),
    )(x, W)
